# baseline (speedup 1.0000x reference)
import functools

import jax
import jax.numpy as jnp
from jax import lax
from jax.experimental import pallas as pl
from jax.experimental.pallas import tpu as pltpu
from jax.experimental.pallas import tpu_sc as plsc

_NC = 2
_NS = 16
_NW = _NC * _NS
_CHUNK = 128


@functools.lru_cache(maxsize=None)
def _make_gather(total, word_dim):
    assert total % (_NW * _CHUNK) == 0
    b_per_w = total // _NW
    n_chunks = b_per_w // _CHUNK
    mesh = plsc.VectorSubcoreMesh(core_axis_name="c", subcore_axis_name="s")

    @functools.partial(
        pl.kernel,
        mesh=mesh,
        compiler_params=pltpu.CompilerParams(use_tc_tiling_on_sc=False),
        out_type=jax.ShapeDtypeStruct((total, word_dim), jnp.float32),
        scratch_types=[
            pltpu.VMEM((n_chunks, _CHUNK), jnp.int32),
            pltpu.VMEM((_CHUNK, word_dim), jnp.float32),
            pltpu.VMEM((_CHUNK, word_dim), jnp.float32),
            pltpu.SemaphoreType.DMA,
            pltpu.SemaphoreType.DMA,
        ],
    )
    def gather_kernel(table_hbm, idx_hbm, out_hbm, idx_v, rows0, rows1, g0, g1):
        wid = lax.axis_index("s") * _NC + lax.axis_index("c")
        base = wid * b_per_w
        pltpu.sync_copy(idx_hbm.at[wid], idx_v)

        pltpu.async_copy(table_hbm.at[idx_v.at[0]], rows0, g0)
        pltpu.async_copy(table_hbm.at[idx_v.at[1]], rows1, g1)

        def wait_gather(rows, sem):
            pltpu.make_async_copy(table_hbm.at[idx_v.at[0]], rows, sem).wait()

        def body(i, carry):
            j0 = i * 2
            wait_gather(rows0, g0)
            pltpu.sync_copy(rows0, out_hbm.at[pl.ds(base + j0 * _CHUNK, _CHUNK)])
            pltpu.async_copy(table_hbm.at[idx_v.at[j0 + 2]], rows0, g0)
            wait_gather(rows1, g1)
            pltpu.sync_copy(
                rows1, out_hbm.at[pl.ds(base + (j0 + 1) * _CHUNK, _CHUNK)])
            pltpu.async_copy(table_hbm.at[idx_v.at[j0 + 3]], rows1, g1)
            return carry

        lax.fori_loop(0, (n_chunks - 2) // 2, body, 0)

        wait_gather(rows0, g0)
        pltpu.sync_copy(
            rows0, out_hbm.at[pl.ds(base + (n_chunks - 2) * _CHUNK, _CHUNK)])
        wait_gather(rows1, g1)
        pltpu.sync_copy(
            rows1, out_hbm.at[pl.ds(base + (n_chunks - 1) * _CHUNK, _CHUNK)])

    return gather_kernel


def kernel(sent, W):
    batch, seq = sent.shape
    word_dim = W.shape[1]
    total = batch * seq
    idx = sent.astype(jnp.int32).reshape(_NW, total // (_NW * _CHUNK), _CHUNK)
    out = _make_gather(total, word_dim)(W, idx)
    return out.reshape(batch, seq, word_dim)


# speedup vs baseline: 1.0158x; 1.0158x over previous
import functools

import jax
import jax.numpy as jnp
from jax import lax
from jax.experimental import pallas as pl
from jax.experimental.pallas import tpu as pltpu
from jax.experimental.pallas import tpu_sc as plsc

_NC = 2
_NS = 16
_NW = _NC * _NS
_CHUNK = 128
_NBUF = 4


@functools.lru_cache(maxsize=None)
def _make_gather(total, word_dim):
    assert total % (_NW * _CHUNK) == 0
    b_per_w = total // _NW
    n_chunks = b_per_w // _CHUNK
    assert n_chunks % _NBUF == 0 and n_chunks >= 2 * _NBUF
    mesh = plsc.VectorSubcoreMesh(core_axis_name="c", subcore_axis_name="s")

    @functools.partial(
        pl.kernel,
        mesh=mesh,
        compiler_params=pltpu.CompilerParams(use_tc_tiling_on_sc=False),
        out_type=jax.ShapeDtypeStruct((total, word_dim), jnp.float32),
        scratch_types=[
            pltpu.VMEM((n_chunks, _CHUNK), jnp.int32),
        ] + [pltpu.VMEM((_CHUNK, word_dim), jnp.float32)] * _NBUF
          + [pltpu.SemaphoreType.DMA] * (2 * _NBUF),
    )
    def gather_kernel(table_hbm, idx_hbm, out_hbm, idx_v, *bufs_and_sems):
        rows = bufs_and_sems[:_NBUF]
        gsem = bufs_and_sems[_NBUF:2 * _NBUF]
        osem = bufs_and_sems[2 * _NBUF:]
        wid = lax.axis_index("s") * _NC + lax.axis_index("c")
        base = wid * b_per_w
        pltpu.sync_copy(idx_hbm.at[wid], idx_v)

        def g_start(j, b):
            pltpu.async_copy(table_hbm.at[idx_v.at[j]], rows[b], gsem[b])

        def g_wait(b):
            pltpu.make_async_copy(
                table_hbm.at[idx_v.at[0]], rows[b], gsem[b]).wait()

        def w_start(j, b):
            pltpu.async_copy(
                rows[b], out_hbm.at[pl.ds(base + j * _CHUNK, _CHUNK)], osem[b])

        def w_wait(b):
            pltpu.make_async_copy(
                rows[b], out_hbm.at[pl.ds(base, _CHUNK)], osem[b]).wait()

        def step(j, b, gather_ahead, first):
            b2 = (b + 2) % _NBUF
            g_wait(b)
            w_start(j, b)
            if gather_ahead:
                if not first:
                    w_wait(b2)
                g_start(j + 2, b2)

        g_start(0, 0)
        g_start(1, 1)
        step(0, 0, True, True)
        step(1, 1, True, True)
        step(2, 2, True, False)
        step(3, 3, True, False)

        def body(i, carry):
            jo = 4 + i * _NBUF
            for k in range(_NBUF):
                step(jo + k, k, True, False)
            return carry

        lax.fori_loop(0, (n_chunks - 2 * _NBUF) // _NBUF, body, 0)

        step(n_chunks - 4, 0, True, False)
        step(n_chunks - 3, 1, True, False)
        step(n_chunks - 2, 2, False, False)
        step(n_chunks - 1, 3, False, False)
        for b in range(_NBUF):
            w_wait(b)

    return gather_kernel


def kernel(sent, W):
    batch, seq = sent.shape
    word_dim = W.shape[1]
    total = batch * seq
    idx = sent.astype(jnp.int32).reshape(_NW, total // (_NW * _CHUNK), _CHUNK)
    out = _make_gather(total, word_dim)(W, idx)
    return out.reshape(batch, seq, word_dim)
